# baseline (speedup 1.0000x reference)
import functools

import jax
import jax.numpy as jnp
import numpy as np
from jax import lax
from jax.experimental import pallas as pl
from jax.experimental.pallas import tpu as pltpu
from jax.experimental.pallas import tpu_sc as plsc

N = 50000
E = 800000
EMB = 64
HALF = EMB // 2
HID = 2 * EMB
L = 5
BN_EPS = 1e-5

NTILES = 16
CH = 256
E_PAD = 819200
CHUNKS = E_PAD // CH
CPT = CHUNKS // NTILES
E_CNT = 819200
TRASH = 64
NS = N + TRASH
ZSH = NS // NTILES
OSH = N // NTILES
CNT_FLAT = NS * 16
CSH = CNT_FLAT // NTILES

_mesh = plsc.VectorSubcoreMesh(core_axis_name="c", subcore_axis_name="s")
_sc_params = pltpu.CompilerParams(use_tc_tiling_on_sc=False)


def _spmv_body(h_hbm, idx_hbm, out_hbm, idx0, idx1, idx2, rows0, rows1,
               rows2, agg_s, isem0, isem1, isem2, gsem0, gsem1, gsem2,
               ssem0, ssem1, ssem2):
    c = lax.axis_index("c")
    s = lax.axis_index("s")
    idx = (idx0, idx1, idx2)
    rows = (rows0, rows1, rows2)
    isem = (isem0, isem1, isem2)
    gsem = (gsem0, gsem1, gsem2)
    ssem = (ssem0, ssem1, ssem2)
    cn = c * N

    def zrow(i, carry):
        rows0[i, pl.ds(0, 16)] = jnp.zeros((16,), jnp.float32)
        rows0[i, pl.ds(16, 16)] = jnp.zeros((16,), jnp.float32)
        return carry

    lax.fori_loop(0, CH, zrow, 0, unroll=8)
    zb = s * ZSH
    for t in range(ZSH // CH):
        pltpu.sync_copy(rows0, agg_s.at[pl.ds(zb + t * CH, CH)])
    zt = ZSH % CH
    pltpu.sync_copy(rows0.at[pl.ds(0, zt)],
                    agg_s.at[pl.ds(zb + ZSH - zt, zt)])
    plsc.subcore_barrier()

    def fire_idx(x, b):
        pltpu.make_async_copy(idx_hbm.at[x * NTILES + s], idx[b],
                              isem[b]).start()

    def wait_idx_fire_gather(x, b):
        pltpu.make_async_copy(idx_hbm.at[x * NTILES + s], idx[b],
                              isem[b]).wait()
        for q in range(CH // 16):
            idx[b][0, pl.ds(q * 16, 16)] = idx[b][0, pl.ds(q * 16, 16)] + cn
        pltpu.make_async_copy(h_hbm.at[idx[b].at[0]], rows[b],
                              gsem[b]).start()

    def wait_gather_fire_scatter(b):
        pltpu.make_async_copy(h_hbm.at[idx[b].at[0]], rows[b],
                              gsem[b]).wait()
        pltpu.make_async_copy(rows[b], agg_s.at[idx[b].at[1]],
                              ssem[b]).start(add=True)

    def wait_scatter(b):
        pltpu.make_async_copy(rows[b], agg_s.at[idx[b].at[1]],
                              ssem[b]).wait()

    assert CPT == 200
    fire_idx(0, 0)
    wait_idx_fire_gather(0, 0)
    fire_idx(1, 1)
    wait_gather_fire_scatter(0)
    wait_idx_fire_gather(1, 1)
    fire_idx(2, 2)

    def tri(jj, carry):
        for p in range(3):
            t = 1 + jj * 3 + p
            b0 = (1 + p) % 3
            wait_gather_fire_scatter(b0)
            wait_idx_fire_gather(t + 1, (2 + p) % 3)
            wait_scatter(p % 3)
            fire_idx(t + 2, p % 3)
        return carry

    lax.fori_loop(0, 65, tri, 0)
    wait_gather_fire_scatter(1)
    wait_idx_fire_gather(197, 2)
    wait_scatter(0)
    fire_idx(198, 0)
    wait_gather_fire_scatter(2)
    wait_idx_fire_gather(198, 0)
    wait_scatter(1)
    fire_idx(199, 1)
    wait_gather_fire_scatter(0)
    wait_idx_fire_gather(199, 1)
    wait_scatter(2)
    wait_gather_fire_scatter(1)
    wait_scatter(0)
    wait_scatter(1)
    plsc.subcore_barrier()
    ob = s * OSH
    nfull = OSH // CH
    ot = OSH % CH
    for t in range(nfull):
        b = t % 2
        if t >= 2:
            pltpu.make_async_copy(
                rows[b], out_hbm.at[pl.ds(cn + ob + (t - 2) * CH, CH)],
                gsem[b]).wait()
        pltpu.sync_copy(agg_s.at[pl.ds(ob + t * CH, CH)], rows[b])
        pltpu.make_async_copy(
            rows[b], out_hbm.at[pl.ds(cn + ob + t * CH, CH)],
            gsem[b]).start()
    for t in (nfull - 2, nfull - 1):
        b = t % 2
        pltpu.make_async_copy(
            rows[b], out_hbm.at[pl.ds(cn + ob + t * CH, CH)],
            gsem[b]).wait()
    pltpu.sync_copy(agg_s.at[pl.ds(ob + OSH - ot, ot)],
                    rows0.at[pl.ds(0, ot)])
    pltpu.sync_copy(rows0.at[pl.ds(0, ot)],
                    out_hbm.at[pl.ds(cn + ob + OSH - ot, ot)])


_spmv = pl.kernel(
    _spmv_body,
    out_type=jax.ShapeDtypeStruct((2 * N, HALF), jnp.float32),
    mesh=_mesh,
    scratch_types=[
        pltpu.VMEM((2, CH), jnp.int32),
        pltpu.VMEM((2, CH), jnp.int32),
        pltpu.VMEM((2, CH), jnp.int32),
        pltpu.VMEM((CH, HALF), jnp.float32),
        pltpu.VMEM((CH, HALF), jnp.float32),
        pltpu.VMEM((CH, HALF), jnp.float32),
        pltpu.VMEM_SHARED((NS, HALF), jnp.float32),
        pltpu.SemaphoreType.DMA,
        pltpu.SemaphoreType.DMA,
        pltpu.SemaphoreType.DMA,
        pltpu.SemaphoreType.DMA,
        pltpu.SemaphoreType.DMA,
        pltpu.SemaphoreType.DMA,
        pltpu.SemaphoreType.DMA,
        pltpu.SemaphoreType.DMA,
        pltpu.SemaphoreType.DMA,
    ],
    compiler_params=_sc_params,
)


def _cnt_body(dst_hbm, code_hbm, out_hbm, dst_v, eidx_v, ones_v, bnc_v, cnt_s,
              osem):
    c = lax.axis_index("c")
    s = lax.axis_index("s")
    wid = s * 2 + c

    def zfill(i, carry):
        bnc_v[pl.ds(i * 16, 16)] = jnp.zeros((16,), jnp.float32)
        return carry

    lax.fori_loop(0, 512, zfill, 0, unroll=8)
    zb = s * CSH
    for t in range(CSH // 8192):
        pltpu.sync_copy(bnc_v, cnt_s.at[pl.ds(zb + t * 8192, 8192)])
    pltpu.sync_copy(bnc_v.at[pl.ds(0, CSH % 8192)],
                    cnt_s.at[pl.ds(zb + CSH - CSH % 8192, CSH % 8192)])

    def ofill(i, carry):
        ones_v[pl.ds(i * 16, 16)] = jnp.ones((16,), jnp.float32)
        return carry

    lax.fori_loop(0, 64, ofill, 0, unroll=8)
    plsc.subcore_barrier()

    def chunk(i, carry):
        base = (i * 32 + wid) * 1024
        pltpu.sync_copy(dst_hbm.at[pl.ds(base, 1024)], dst_v)
        pltpu.sync_copy(code_hbm.at[pl.ds(base, 1024)], eidx_v)
        for q in range(64):
            eidx_v[pl.ds(q * 16, 16)] = (dst_v[pl.ds(q * 16, 16)] * 16
                                         + eidx_v[pl.ds(q * 16, 16)])
        pltpu.sync_copy(ones_v, cnt_s.at[eidx_v], add=True)
        return carry

    lax.fori_loop(0, E_CNT // 1024 // 32, chunk, 0)
    plsc.subcore_barrier()
    cb = s * CSH
    for t in range(6):
        pltpu.sync_copy(cnt_s.at[pl.ds(cb + t * 8192, 8192)], bnc_v)
        pltpu.sync_copy(bnc_v,
                        out_hbm.at[pl.ds(c * CNT_FLAT + cb + t * 8192, 8192)])
    pltpu.sync_copy(cnt_s.at[pl.ds(cb + 49152, 912)], bnc_v.at[pl.ds(0, 912)])
    pltpu.sync_copy(bnc_v.at[pl.ds(0, 912)],
                    out_hbm.at[pl.ds(c * CNT_FLAT + cb + 49152, 912)])


_cnt = pl.kernel(
    _cnt_body,
    out_type=jax.ShapeDtypeStruct((2 * CNT_FLAT,), jnp.float32),
    mesh=_mesh,
    scratch_types=[
        pltpu.VMEM((1024,), jnp.int32),
        pltpu.VMEM((1024,), jnp.int32),
        pltpu.VMEM((1024,), jnp.float32),
        pltpu.VMEM((8192,), jnp.float32),
        pltpu.VMEM_SHARED((CNT_FLAT,), jnp.float32),
        pltpu.SemaphoreType.DMA,
    ],
    compiler_params=_sc_params,
)

_BN = 2000
_NB = N // _BN


def _init_body(code_ref, t0_ref, o_ref):
    code = code_ref[0, 0, :]
    oh = (code[:, None] == lax.broadcasted_iota(jnp.int32, (_BN, 16), 1))
    h0 = jnp.dot(oh.astype(jnp.float32), t0_ref[...],
                 preferred_element_type=jnp.float32)
    o_ref[0] = h0[:, :HALF]
    o_ref[1] = h0[:, HALF:]


def _mlp_body(aA_ref, aB_ref, c0_ref, c1_ref, cw1_ref, w1a_ref, w1b_ref,
              w2_ref, b1_ref, b2_ref, g_ref, bt_ref, o_ref, *, relu):
    cnt = c0_ref[0] + c1_ref[0]
    z = jnp.dot(aA_ref[...], w1a_ref[...], preferred_element_type=jnp.float32)
    z = z + jnp.dot(aB_ref[...], w1b_ref[...],
                    preferred_element_type=jnp.float32)
    z = z + jnp.dot(cnt, cw1_ref[...], preferred_element_type=jnp.float32)
    hid = jnp.maximum(z + b1_ref[...], 0.0)
    y = jnp.dot(hid, w2_ref[...], preferred_element_type=jnp.float32)
    y = (y + b2_ref[...]) * g_ref[...] + bt_ref[...]
    if relu:
        y = jnp.maximum(y, 0.0)
    o_ref[0] = y[:, :HALF]
    o_ref[1] = y[:, HALF:]


def _mlp_final_body(aA_ref, aB_ref, c0_ref, c1_ref, cw1_ref, w1a_ref,
                    w1b_ref, w2_ref, b1_ref, b2_ref, g_ref, bt_ref, wp_ref,
                    bp_ref, o_ref, acc_ref):
    i = pl.program_id(0)

    @pl.when(i == 0)
    def _():
        acc_ref[...] = jnp.zeros_like(acc_ref)

    cnt = c0_ref[0] + c1_ref[0]
    z = jnp.dot(aA_ref[...], w1a_ref[...], preferred_element_type=jnp.float32)
    z = z + jnp.dot(aB_ref[...], w1b_ref[...],
                    preferred_element_type=jnp.float32)
    z = z + jnp.dot(cnt, cw1_ref[...], preferred_element_type=jnp.float32)
    hid = jnp.maximum(z + b1_ref[...], 0.0)
    y = jnp.dot(hid, w2_ref[...], preferred_element_type=jnp.float32)
    y = (y + b2_ref[...]) * g_ref[...] + bt_ref[...]
    acc_ref[0:1, 0:EMB] += jnp.sum(y, axis=0, keepdims=True)

    @pl.when(i == _NB - 1)
    def _():
        tot = acc_ref[0:1, 0:EMB]
        o_ref[...] = (jnp.sum(tot * wp_ref[...], axis=1, keepdims=True)
                      * (1.0 / N) + bp_ref[...])


def _full(shape):
    nd = len(shape)
    return pl.BlockSpec(shape, lambda i: (0,) * nd)


def _mlp_call(aggsc, cntf, cw1, w1a, w1b, w2, b1, b2, gsc, beta, relu):
    return pl.pallas_call(
        functools.partial(_mlp_body, relu=relu),
        grid=(_NB,),
        in_specs=[
            pl.BlockSpec((_BN, HALF), lambda i: (i, 0)),
            pl.BlockSpec((_BN, HALF), lambda i: (_NB + i, 0)),
            pl.BlockSpec((1, _BN, 16), lambda i: (0, i, 0)),
            pl.BlockSpec((1, _BN, 16), lambda i: (1, i, 0)),
            _full((16, HID)),
            _full((HALF, HID)),
            _full((HALF, HID)),
            _full((HID, EMB)),
            _full((1, HID)),
            _full((1, EMB)),
            _full((1, EMB)),
            _full((1, EMB)),
        ],
        out_specs=pl.BlockSpec((2, _BN, HALF), lambda i: (0, i, 0)),
        out_shape=jax.ShapeDtypeStruct((2, N, HALF), jnp.float32),
    )(aggsc, aggsc, cntf, cntf, cw1, w1a, w1b, w2, b1, b2, gsc, beta)


def _mlp_final_call(aggsc, cntf, cw1, w1a, w1b, w2, b1, b2, gsc, beta,
                    wp, bp):
    return pl.pallas_call(
        _mlp_final_body,
        grid=(_NB,),
        in_specs=[
            pl.BlockSpec((_BN, HALF), lambda i: (i, 0)),
            pl.BlockSpec((_BN, HALF), lambda i: (_NB + i, 0)),
            pl.BlockSpec((1, _BN, 16), lambda i: (0, i, 0)),
            pl.BlockSpec((1, _BN, 16), lambda i: (1, i, 0)),
            _full((16, HID)),
            _full((HALF, HID)),
            _full((HALF, HID)),
            _full((HID, EMB)),
            _full((1, HID)),
            _full((1, EMB)),
            _full((1, EMB)),
            _full((1, EMB)),
            _full((1, EMB)),
            _full((1, 1)),
        ],
        out_specs=_full((1, 1)),
        out_shape=jax.ShapeDtypeStruct((1, 1), jnp.float32),
        scratch_shapes=[pltpu.VMEM((8, 128), jnp.float32)],
    )(aggsc, aggsc, cntf, cntf, cw1, w1a, w1b, w2, b1, b2, gsc, beta, wp, bp)


def kernel(g, x, w, node_emb0, node_emb1, edge_emb0, edge_emb1, W1, b1, W2,
           b2, gamma, beta, Wp, bp):
    f32 = jnp.float32
    src = g[0].astype(jnp.int32)
    dst = g[1].astype(jnp.int32)
    pad = E_PAD - E
    ar = jnp.arange(pad, dtype=jnp.int32)
    srcf = jnp.concatenate([src, ar % 997])
    dstf = jnp.concatenate([dst, N + (ar % TRASH)])
    idxc = jnp.concatenate([srcf.reshape(CHUNKS, 1, CH),
                            dstf.reshape(CHUNKS, 1, CH)], axis=1)
    code_e = (w[:, 0] * 3 + w[:, 1]).astype(jnp.int32)
    codep = jnp.concatenate([code_e, jnp.zeros((pad,), jnp.int32)])
    code_n = (x[:, 0] * 3 + x[:, 1]).astype(jnp.int32).reshape(_NB, 1, _BN)

    t9 = (node_emb0[:3, None, :] + node_emb1[None, :3, :]).reshape(9, EMB)
    t16 = jnp.concatenate([t9, jnp.zeros((7, EMB), f32)])
    e9 = (edge_emb0[:, :3, None, :]
          + edge_emb1[:, None, :3, :]).reshape(L, 9, EMB)
    e16 = jnp.concatenate([e9, jnp.zeros((L, 7, EMB), f32)], axis=1)

    inv_std = np.float32(1.0 / np.sqrt(1.0 + BN_EPS))

    cntf = _cnt(dstf, codep).reshape(2, NS, 16)

    h = pl.pallas_call(
        _init_body,
        grid=(_NB,),
        in_specs=[
            pl.BlockSpec((1, 1, _BN), lambda i: (i, 0, 0)),
            _full((16, EMB)),
        ],
        out_specs=pl.BlockSpec((2, _BN, HALF), lambda i: (0, i, 0)),
        out_shape=jax.ShapeDtypeStruct((2, N, HALF), f32),
    )(code_n, t16)

    out = None
    for l in range(L):
        agg = _spmv(h.reshape(2 * N, HALF), idxc)
        cw1 = jnp.dot(e16[l], W1[l])
        args = (agg, cntf, cw1,
                W1[l][:HALF], W1[l][HALF:], W2[l],
                b1[l].reshape(1, HID), b2[l].reshape(1, EMB),
                (gamma[l] * inv_std).reshape(1, EMB), beta[l].reshape(1, EMB))
        if l < L - 1:
            h = _mlp_call(*args, relu=True)
        else:
            out = _mlp_final_call(*args, Wp.reshape(1, EMB),
                                  bp.reshape(1, 1))
    return out


# speedup vs baseline: 1.1215x; 1.1215x over previous
import functools

import jax
import jax.numpy as jnp
import numpy as np
from jax import lax
from jax.experimental import pallas as pl
from jax.experimental.pallas import tpu as pltpu
from jax.experimental.pallas import tpu_sc as plsc

N = 50000
E = 800000
EMB = 64
HALF = EMB // 2
HID = 2 * EMB
L = 5
BN_EPS = 1e-5

NTILES = 16
CH = 448
E_PAD = 802816
CHUNKS = E_PAD // CH
CPT = CHUNKS // NTILES
CCH = 512
TRASH = 64
NS = N + TRASH
ZSH = NS // NTILES
OSH = N // NTILES
CNT_FLAT = NS * 16
CSH = CNT_FLAT // NTILES

_mesh = plsc.VectorSubcoreMesh(core_axis_name="c", subcore_axis_name="s")
_sc_params = pltpu.CompilerParams(use_tc_tiling_on_sc=False)


def _spmv_body(h_hbm, idx_hbm, out_hbm, idx0, idx1, rows0, rows1, agg_s,
               gsem0, gsem1):
    c = lax.axis_index("c")
    s = lax.axis_index("s")
    idx = (idx0, idx1)
    rows = (rows0, rows1)
    gsem = (gsem0, gsem1)
    cn = c * N

    def zrow(i, carry):
        rows0[i, pl.ds(0, 16)] = jnp.zeros((16,), jnp.float32)
        rows0[i, pl.ds(16, 16)] = jnp.zeros((16,), jnp.float32)
        return carry

    lax.fori_loop(0, CH, zrow, 0, unroll=8)
    zb = s * ZSH
    for t in range(ZSH // CH):
        pltpu.sync_copy(rows0, agg_s.at[pl.ds(zb + t * CH, CH)])
    zt = ZSH % CH
    pltpu.sync_copy(rows0.at[pl.ds(0, zt)],
                    agg_s.at[pl.ds(zb + ZSH - zt, zt)])
    plsc.subcore_barrier()

    def load_and_fire(j, b):
        chunk = j * NTILES + s
        pltpu.sync_copy(idx_hbm.at[chunk], idx[b])
        for q in range(CH // 16):
            idx[b][0, pl.ds(q * 16, 16)] = idx[b][0, pl.ds(q * 16, 16)] + cn
        pltpu.make_async_copy(h_hbm.at[idx[b].at[0]], rows[b],
                              gsem[b]).start()

    def drain_and_scatter(b):
        pltpu.make_async_copy(h_hbm.at[idx[b].at[0]], rows[b],
                              gsem[b]).wait()
        pltpu.sync_copy(rows[b], agg_s.at[idx[b].at[1]], add=True)

    load_and_fire(0, 0)
    load_and_fire(1, 1)

    def pair(jj, carry):
        for b in range(2):
            j = jj * 2 + b
            drain_and_scatter(b)

            @pl.when(j + 2 < CPT)
            def _():
                load_and_fire(j + 2, b)

        return carry

    lax.fori_loop(0, CPT // 2, pair, 0)
    plsc.subcore_barrier()
    ob = s * OSH
    nfull = OSH // CH
    ot = OSH % CH
    for t in range(nfull):
        b = t % 2
        if t >= 2:
            pltpu.make_async_copy(
                rows[b], out_hbm.at[pl.ds(cn + ob + (t - 2) * CH, CH)],
                gsem[b]).wait()
        pltpu.sync_copy(agg_s.at[pl.ds(ob + t * CH, CH)], rows[b])
        pltpu.make_async_copy(
            rows[b], out_hbm.at[pl.ds(cn + ob + t * CH, CH)],
            gsem[b]).start()
    for t in (nfull - 2, nfull - 1):
        b = t % 2
        pltpu.make_async_copy(
            rows[b], out_hbm.at[pl.ds(cn + ob + t * CH, CH)],
            gsem[b]).wait()
    pltpu.sync_copy(agg_s.at[pl.ds(ob + OSH - ot, ot)],
                    rows0.at[pl.ds(0, ot)])
    pltpu.sync_copy(rows0.at[pl.ds(0, ot)],
                    out_hbm.at[pl.ds(cn + ob + OSH - ot, ot)])


_spmv = pl.kernel(
    _spmv_body,
    out_type=jax.ShapeDtypeStruct((2 * N, HALF), jnp.float32),
    mesh=_mesh,
    scratch_types=[
        pltpu.VMEM((2, CH), jnp.int32),
        pltpu.VMEM((2, CH), jnp.int32),
        pltpu.VMEM((CH, HALF), jnp.float32),
        pltpu.VMEM((CH, HALF), jnp.float32),
        pltpu.VMEM_SHARED((NS, HALF), jnp.float32),
        pltpu.SemaphoreType.DMA,
        pltpu.SemaphoreType.DMA,
    ],
    compiler_params=_sc_params,
)


def _cnt_body(dst_hbm, code_hbm, out_hbm, dst_v, eidx_v, ones_v, bnc_v, cnt_s,
              osem):
    c = lax.axis_index("c")
    s = lax.axis_index("s")
    wid = s * 2 + c

    def zfill(i, carry):
        bnc_v[pl.ds(i * 16, 16)] = jnp.zeros((16,), jnp.float32)
        return carry

    lax.fori_loop(0, 512, zfill, 0, unroll=8)
    zb = s * CSH
    for t in range(CSH // 8192):
        pltpu.sync_copy(bnc_v, cnt_s.at[pl.ds(zb + t * 8192, 8192)])
    pltpu.sync_copy(bnc_v.at[pl.ds(0, CSH % 8192)],
                    cnt_s.at[pl.ds(zb + CSH - CSH % 8192, CSH % 8192)])

    def ofill(i, carry):
        ones_v[pl.ds(i * 16, 16)] = jnp.ones((16,), jnp.float32)
        return carry

    lax.fori_loop(0, CCH // 16, ofill, 0, unroll=8)
    plsc.subcore_barrier()

    def chunk(i, carry):
        base = (i * 32 + wid) * CCH
        pltpu.sync_copy(dst_hbm.at[pl.ds(base, CCH)], dst_v)
        pltpu.sync_copy(code_hbm.at[pl.ds(base, CCH)], eidx_v)
        for q in range(CCH // 16):
            eidx_v[pl.ds(q * 16, 16)] = (dst_v[pl.ds(q * 16, 16)] * 16
                                         + eidx_v[pl.ds(q * 16, 16)])
        pltpu.sync_copy(ones_v, cnt_s.at[eidx_v], add=True)
        return carry

    lax.fori_loop(0, E_PAD // CCH // 32, chunk, 0)
    plsc.subcore_barrier()
    cb = s * CSH
    for t in range(6):
        pltpu.sync_copy(cnt_s.at[pl.ds(cb + t * 8192, 8192)], bnc_v)
        pltpu.sync_copy(bnc_v,
                        out_hbm.at[pl.ds(c * CNT_FLAT + cb + t * 8192, 8192)])
    pltpu.sync_copy(cnt_s.at[pl.ds(cb + 49152, 912)], bnc_v.at[pl.ds(0, 912)])
    pltpu.sync_copy(bnc_v.at[pl.ds(0, 912)],
                    out_hbm.at[pl.ds(c * CNT_FLAT + cb + 49152, 912)])


_cnt = pl.kernel(
    _cnt_body,
    out_type=jax.ShapeDtypeStruct((2 * CNT_FLAT,), jnp.float32),
    mesh=_mesh,
    scratch_types=[
        pltpu.VMEM((CCH,), jnp.int32),
        pltpu.VMEM((CCH,), jnp.int32),
        pltpu.VMEM((CCH,), jnp.float32),
        pltpu.VMEM((8192,), jnp.float32),
        pltpu.VMEM_SHARED((CNT_FLAT,), jnp.float32),
        pltpu.SemaphoreType.DMA,
    ],
    compiler_params=_sc_params,
)

_BN = 2000
_NB = N // _BN


def _init_body(code_ref, t0_ref, o_ref):
    code = code_ref[0, 0, :]
    oh = (code[:, None] == lax.broadcasted_iota(jnp.int32, (_BN, 16), 1))
    h0 = jnp.dot(oh.astype(jnp.float32), t0_ref[...],
                 preferred_element_type=jnp.float32)
    o_ref[0] = h0[:, :HALF]
    o_ref[1] = h0[:, HALF:]


def _mlp_body(aA_ref, aB_ref, c0_ref, c1_ref, cw1_ref, w1a_ref, w1b_ref,
              w2_ref, b1_ref, b2_ref, g_ref, bt_ref, o_ref, *, relu):
    cnt = c0_ref[0] + c1_ref[0]
    z = jnp.dot(aA_ref[...], w1a_ref[...], preferred_element_type=jnp.float32)
    z = z + jnp.dot(aB_ref[...], w1b_ref[...],
                    preferred_element_type=jnp.float32)
    z = z + jnp.dot(cnt, cw1_ref[...], preferred_element_type=jnp.float32)
    hid = jnp.maximum(z + b1_ref[...], 0.0)
    y = jnp.dot(hid, w2_ref[...], preferred_element_type=jnp.float32)
    y = (y + b2_ref[...]) * g_ref[...] + bt_ref[...]
    if relu:
        y = jnp.maximum(y, 0.0)
    o_ref[0] = y[:, :HALF]
    o_ref[1] = y[:, HALF:]


def _mlp_final_body(aA_ref, aB_ref, c0_ref, c1_ref, cw1_ref, w1a_ref,
                    w1b_ref, w2_ref, b1_ref, b2_ref, g_ref, bt_ref, wp_ref,
                    bp_ref, o_ref, acc_ref):
    i = pl.program_id(0)

    @pl.when(i == 0)
    def _():
        acc_ref[...] = jnp.zeros_like(acc_ref)

    cnt = c0_ref[0] + c1_ref[0]
    z = jnp.dot(aA_ref[...], w1a_ref[...], preferred_element_type=jnp.float32)
    z = z + jnp.dot(aB_ref[...], w1b_ref[...],
                    preferred_element_type=jnp.float32)
    z = z + jnp.dot(cnt, cw1_ref[...], preferred_element_type=jnp.float32)
    hid = jnp.maximum(z + b1_ref[...], 0.0)
    y = jnp.dot(hid, w2_ref[...], preferred_element_type=jnp.float32)
    y = (y + b2_ref[...]) * g_ref[...] + bt_ref[...]
    acc_ref[0:1, 0:EMB] += jnp.sum(y, axis=0, keepdims=True)

    @pl.when(i == _NB - 1)
    def _():
        tot = acc_ref[0:1, 0:EMB]
        o_ref[...] = (jnp.sum(tot * wp_ref[...], axis=1, keepdims=True)
                      * (1.0 / N) + bp_ref[...])


def _full(shape):
    nd = len(shape)
    return pl.BlockSpec(shape, lambda i: (0,) * nd)


def _mlp_call(aggsc, cntf, cw1, w1a, w1b, w2, b1, b2, gsc, beta, relu):
    return pl.pallas_call(
        functools.partial(_mlp_body, relu=relu),
        grid=(_NB,),
        in_specs=[
            pl.BlockSpec((_BN, HALF), lambda i: (i, 0)),
            pl.BlockSpec((_BN, HALF), lambda i: (_NB + i, 0)),
            pl.BlockSpec((1, _BN, 16), lambda i: (0, i, 0)),
            pl.BlockSpec((1, _BN, 16), lambda i: (1, i, 0)),
            _full((16, HID)),
            _full((HALF, HID)),
            _full((HALF, HID)),
            _full((HID, EMB)),
            _full((1, HID)),
            _full((1, EMB)),
            _full((1, EMB)),
            _full((1, EMB)),
        ],
        out_specs=pl.BlockSpec((2, _BN, HALF), lambda i: (0, i, 0)),
        out_shape=jax.ShapeDtypeStruct((2, N, HALF), jnp.float32),
    )(aggsc, aggsc, cntf, cntf, cw1, w1a, w1b, w2, b1, b2, gsc, beta)


def _mlp_final_call(aggsc, cntf, cw1, w1a, w1b, w2, b1, b2, gsc, beta,
                    wp, bp):
    return pl.pallas_call(
        _mlp_final_body,
        grid=(_NB,),
        in_specs=[
            pl.BlockSpec((_BN, HALF), lambda i: (i, 0)),
            pl.BlockSpec((_BN, HALF), lambda i: (_NB + i, 0)),
            pl.BlockSpec((1, _BN, 16), lambda i: (0, i, 0)),
            pl.BlockSpec((1, _BN, 16), lambda i: (1, i, 0)),
            _full((16, HID)),
            _full((HALF, HID)),
            _full((HALF, HID)),
            _full((HID, EMB)),
            _full((1, HID)),
            _full((1, EMB)),
            _full((1, EMB)),
            _full((1, EMB)),
            _full((1, EMB)),
            _full((1, 1)),
        ],
        out_specs=_full((1, 1)),
        out_shape=jax.ShapeDtypeStruct((1, 1), jnp.float32),
        scratch_shapes=[pltpu.VMEM((8, 128), jnp.float32)],
    )(aggsc, aggsc, cntf, cntf, cw1, w1a, w1b, w2, b1, b2, gsc, beta, wp, bp)


def kernel(g, x, w, node_emb0, node_emb1, edge_emb0, edge_emb1, W1, b1, W2,
           b2, gamma, beta, Wp, bp):
    f32 = jnp.float32
    src = g[0].astype(jnp.int32)
    dst = g[1].astype(jnp.int32)
    pad = E_PAD - E
    ar = jnp.arange(pad, dtype=jnp.int32)
    srcf = jnp.concatenate([src, ar % 997])
    dstf = jnp.concatenate([dst, N + (ar % TRASH)])
    idxc = jnp.concatenate([srcf.reshape(CHUNKS, 1, CH),
                            dstf.reshape(CHUNKS, 1, CH)], axis=1)
    code_e = (w[:, 0] * 3 + w[:, 1]).astype(jnp.int32)
    codep = jnp.concatenate([code_e, jnp.zeros((pad,), jnp.int32)])
    code_n = (x[:, 0] * 3 + x[:, 1]).astype(jnp.int32).reshape(_NB, 1, _BN)

    t9 = (node_emb0[:3, None, :] + node_emb1[None, :3, :]).reshape(9, EMB)
    t16 = jnp.concatenate([t9, jnp.zeros((7, EMB), f32)])
    e9 = (edge_emb0[:, :3, None, :]
          + edge_emb1[:, None, :3, :]).reshape(L, 9, EMB)
    e16 = jnp.concatenate([e9, jnp.zeros((L, 7, EMB), f32)], axis=1)

    inv_std = np.float32(1.0 / np.sqrt(1.0 + BN_EPS))

    cntf = _cnt(dstf, codep).reshape(2, NS, 16)

    h = pl.pallas_call(
        _init_body,
        grid=(_NB,),
        in_specs=[
            pl.BlockSpec((1, 1, _BN), lambda i: (i, 0, 0)),
            _full((16, EMB)),
        ],
        out_specs=pl.BlockSpec((2, _BN, HALF), lambda i: (0, i, 0)),
        out_shape=jax.ShapeDtypeStruct((2, N, HALF), f32),
    )(code_n, t16)

    out = None
    for l in range(L):
        agg = _spmv(h.reshape(2 * N, HALF), idxc)
        cw1 = jnp.dot(e16[l], W1[l])
        args = (agg, cntf, cw1,
                W1[l][:HALF], W1[l][HALF:], W2[l],
                b1[l].reshape(1, HID), b2[l].reshape(1, EMB),
                (gamma[l] * inv_std).reshape(1, EMB), beta[l].reshape(1, EMB))
        if l < L - 1:
            h = _mlp_call(*args, relu=True)
        else:
            out = _mlp_final_call(*args, Wp.reshape(1, EMB),
                                  bp.reshape(1, 1))
    return out


# speedup vs baseline: 1.1267x; 1.0046x over previous
import functools

import jax
import jax.numpy as jnp
import numpy as np
from jax import lax
from jax.experimental import pallas as pl
from jax.experimental.pallas import tpu as pltpu
from jax.experimental.pallas import tpu_sc as plsc

N = 50000
E = 800000
EMB = 64
HALF = EMB // 2
HID = 2 * EMB
L = 5
BN_EPS = 1e-5

NTILES = 16
CH = 448
E_PAD = 802816
CHUNKS = E_PAD // CH
CPT = CHUNKS // NTILES
CCH = 512
TRASH = 64
NS = N + TRASH
ZSH = NS // NTILES
OSH = N // NTILES
CNT_FLAT = NS * 16
CSH = CNT_FLAT // NTILES

_mesh = plsc.VectorSubcoreMesh(core_axis_name="c", subcore_axis_name="s")
_sc_params = pltpu.CompilerParams(use_tc_tiling_on_sc=False)


def _spmv_body(h_hbm, idx_hbm, out_hbm, idx0, idx1, rows0, rows1, agg_s,
               gsem0, gsem1):
    c = lax.axis_index("c")
    s = lax.axis_index("s")
    idx = (idx0, idx1)
    rows = (rows0, rows1)
    gsem = (gsem0, gsem1)
    cn = c * N

    def zrow(i, carry):
        rows0[i, pl.ds(0, 16)] = jnp.zeros((16,), jnp.float32)
        rows0[i, pl.ds(16, 16)] = jnp.zeros((16,), jnp.float32)
        return carry

    lax.fori_loop(0, CH, zrow, 0, unroll=8)
    zb = s * ZSH
    for t in range(ZSH // CH):
        pltpu.sync_copy(rows0, agg_s.at[pl.ds(zb + t * CH, CH)])
    zt = ZSH % CH
    pltpu.sync_copy(rows0.at[pl.ds(0, zt)],
                    agg_s.at[pl.ds(zb + ZSH - zt, zt)])
    plsc.subcore_barrier()

    def load_and_fire(j, b):
        chunk = j * NTILES + s
        pltpu.sync_copy(idx_hbm.at[chunk], idx[b])
        for q in range(CH // 16):
            idx[b][0, pl.ds(q * 16, 16)] = idx[b][0, pl.ds(q * 16, 16)] + cn
        pltpu.make_async_copy(h_hbm.at[idx[b].at[0]], rows[b],
                              gsem[b]).start()

    def drain_and_scatter(b):
        pltpu.make_async_copy(h_hbm.at[idx[b].at[0]], rows[b],
                              gsem[b]).wait()
        pltpu.sync_copy(rows[b], agg_s.at[idx[b].at[1]], add=True)

    load_and_fire(0, 0)
    load_and_fire(1, 1)

    def pair(jj, carry):
        for b in range(2):
            j = jj * 2 + b
            drain_and_scatter(b)

            @pl.when(j + 2 < CPT)
            def _():
                load_and_fire(j + 2, b)

        return carry

    lax.fori_loop(0, CPT // 2, pair, 0)
    plsc.subcore_barrier()
    ob = s * OSH
    nfull = OSH // CH
    ot = OSH % CH
    for t in range(nfull):
        b = t % 2
        if t >= 2:
            pltpu.make_async_copy(
                rows[b], out_hbm.at[pl.ds(cn + ob + (t - 2) * CH, CH)],
                gsem[b]).wait()
        pltpu.sync_copy(agg_s.at[pl.ds(ob + t * CH, CH)], rows[b])
        pltpu.make_async_copy(
            rows[b], out_hbm.at[pl.ds(cn + ob + t * CH, CH)],
            gsem[b]).start()
    for t in (nfull - 2, nfull - 1):
        b = t % 2
        pltpu.make_async_copy(
            rows[b], out_hbm.at[pl.ds(cn + ob + t * CH, CH)],
            gsem[b]).wait()
    pltpu.sync_copy(agg_s.at[pl.ds(ob + OSH - ot, ot)],
                    rows0.at[pl.ds(0, ot)])
    pltpu.sync_copy(rows0.at[pl.ds(0, ot)],
                    out_hbm.at[pl.ds(cn + ob + OSH - ot, ot)])


_spmv = pl.kernel(
    _spmv_body,
    out_type=jax.ShapeDtypeStruct((2 * N, HALF), jnp.float32),
    mesh=_mesh,
    scratch_types=[
        pltpu.VMEM((2, CH), jnp.int32),
        pltpu.VMEM((2, CH), jnp.int32),
        pltpu.VMEM((CH, HALF), jnp.float32),
        pltpu.VMEM((CH, HALF), jnp.float32),
        pltpu.VMEM_SHARED((NS, HALF), jnp.float32),
        pltpu.SemaphoreType.DMA,
        pltpu.SemaphoreType.DMA,
    ],
    compiler_params=_sc_params,
)


def _cnt_body(dst_hbm, code_hbm, out_hbm, dst_v, eidx_v, ones_v, bnc_v, cnt_s,
              osem):
    c = lax.axis_index("c")
    s = lax.axis_index("s")
    wid = s * 2 + c

    def zfill(i, carry):
        bnc_v[pl.ds(i * 16, 16)] = jnp.zeros((16,), jnp.float32)
        return carry

    lax.fori_loop(0, 512, zfill, 0, unroll=8)
    zb = s * CSH
    for t in range(CSH // 8192):
        pltpu.sync_copy(bnc_v, cnt_s.at[pl.ds(zb + t * 8192, 8192)])
    pltpu.sync_copy(bnc_v.at[pl.ds(0, CSH % 8192)],
                    cnt_s.at[pl.ds(zb + CSH - CSH % 8192, CSH % 8192)])

    def ofill(i, carry):
        ones_v[pl.ds(i * 16, 16)] = jnp.ones((16,), jnp.float32)
        return carry

    lax.fori_loop(0, CCH // 16, ofill, 0, unroll=8)
    plsc.subcore_barrier()

    def chunk(i, carry):
        base = (i * 32 + wid) * CCH
        pltpu.sync_copy(dst_hbm.at[pl.ds(base, CCH)], dst_v)
        pltpu.sync_copy(code_hbm.at[pl.ds(base, CCH)], eidx_v)
        for q in range(CCH // 16):
            eidx_v[pl.ds(q * 16, 16)] = (dst_v[pl.ds(q * 16, 16)] * 16
                                         + eidx_v[pl.ds(q * 16, 16)])
        pltpu.sync_copy(ones_v, cnt_s.at[eidx_v], add=True)
        return carry

    lax.fori_loop(0, E_PAD // CCH // 32, chunk, 0)
    plsc.subcore_barrier()
    cb = s * CSH
    for t in range(6):
        pltpu.sync_copy(cnt_s.at[pl.ds(cb + t * 8192, 8192)], bnc_v)
        pltpu.sync_copy(bnc_v,
                        out_hbm.at[pl.ds(c * CNT_FLAT + cb + t * 8192, 8192)])
    pltpu.sync_copy(cnt_s.at[pl.ds(cb + 49152, 912)], bnc_v.at[pl.ds(0, 912)])
    pltpu.sync_copy(bnc_v.at[pl.ds(0, 912)],
                    out_hbm.at[pl.ds(c * CNT_FLAT + cb + 49152, 912)])


_cnt = pl.kernel(
    _cnt_body,
    out_type=jax.ShapeDtypeStruct((2 * CNT_FLAT,), jnp.float32),
    mesh=_mesh,
    scratch_types=[
        pltpu.VMEM((CCH,), jnp.int32),
        pltpu.VMEM((CCH,), jnp.int32),
        pltpu.VMEM((CCH,), jnp.float32),
        pltpu.VMEM((8192,), jnp.float32),
        pltpu.VMEM_SHARED((CNT_FLAT,), jnp.float32),
        pltpu.SemaphoreType.DMA,
    ],
    compiler_params=_sc_params,
)

_BN = 5000
_NB = N // _BN


def _init_body(code_ref, t0_ref, o_ref):
    code = code_ref[0, 0, :]
    oh = (code[:, None] == lax.broadcasted_iota(jnp.int32, (_BN, 16), 1))
    h0 = jnp.dot(oh.astype(jnp.float32), t0_ref[...],
                 preferred_element_type=jnp.float32)
    o_ref[0] = h0[:, :HALF]
    o_ref[1] = h0[:, HALF:]


def _mlp_body(aA_ref, aB_ref, c0_ref, c1_ref, cw1_ref, w1a_ref, w1b_ref,
              w2_ref, b1_ref, b2_ref, g_ref, bt_ref, o_ref, *, relu):
    cnt = c0_ref[0] + c1_ref[0]
    z = jnp.dot(aA_ref[...], w1a_ref[...], preferred_element_type=jnp.float32)
    z = z + jnp.dot(aB_ref[...], w1b_ref[...],
                    preferred_element_type=jnp.float32)
    z = z + jnp.dot(cnt, cw1_ref[...], preferred_element_type=jnp.float32)
    hid = jnp.maximum(z + b1_ref[...], 0.0)
    y = jnp.dot(hid, w2_ref[...], preferred_element_type=jnp.float32)
    y = (y + b2_ref[...]) * g_ref[...] + bt_ref[...]
    if relu:
        y = jnp.maximum(y, 0.0)
    o_ref[0] = y[:, :HALF]
    o_ref[1] = y[:, HALF:]


def _mlp_final_body(aA_ref, aB_ref, c0_ref, c1_ref, cw1_ref, w1a_ref,
                    w1b_ref, w2_ref, b1_ref, b2_ref, g_ref, bt_ref, wp_ref,
                    bp_ref, o_ref, acc_ref):
    i = pl.program_id(0)

    @pl.when(i == 0)
    def _():
        acc_ref[...] = jnp.zeros_like(acc_ref)

    cnt = c0_ref[0] + c1_ref[0]
    z = jnp.dot(aA_ref[...], w1a_ref[...], preferred_element_type=jnp.float32)
    z = z + jnp.dot(aB_ref[...], w1b_ref[...],
                    preferred_element_type=jnp.float32)
    z = z + jnp.dot(cnt, cw1_ref[...], preferred_element_type=jnp.float32)
    hid = jnp.maximum(z + b1_ref[...], 0.0)
    y = jnp.dot(hid, w2_ref[...], preferred_element_type=jnp.float32)
    y = (y + b2_ref[...]) * g_ref[...] + bt_ref[...]
    acc_ref[0:1, 0:EMB] += jnp.sum(y, axis=0, keepdims=True)

    @pl.when(i == _NB - 1)
    def _():
        tot = acc_ref[0:1, 0:EMB]
        o_ref[...] = (jnp.sum(tot * wp_ref[...], axis=1, keepdims=True)
                      * (1.0 / N) + bp_ref[...])


def _full(shape):
    nd = len(shape)
    return pl.BlockSpec(shape, lambda i: (0,) * nd)


def _mlp_call(aggsc, cntf, cw1, w1a, w1b, w2, b1, b2, gsc, beta, relu):
    return pl.pallas_call(
        functools.partial(_mlp_body, relu=relu),
        grid=(_NB,),
        in_specs=[
            pl.BlockSpec((_BN, HALF), lambda i: (i, 0)),
            pl.BlockSpec((_BN, HALF), lambda i: (_NB + i, 0)),
            pl.BlockSpec((1, _BN, 16), lambda i: (0, i, 0)),
            pl.BlockSpec((1, _BN, 16), lambda i: (1, i, 0)),
            _full((16, HID)),
            _full((HALF, HID)),
            _full((HALF, HID)),
            _full((HID, EMB)),
            _full((1, HID)),
            _full((1, EMB)),
            _full((1, EMB)),
            _full((1, EMB)),
        ],
        out_specs=pl.BlockSpec((2, _BN, HALF), lambda i: (0, i, 0)),
        out_shape=jax.ShapeDtypeStruct((2, N, HALF), jnp.float32),
    )(aggsc, aggsc, cntf, cntf, cw1, w1a, w1b, w2, b1, b2, gsc, beta)


def _mlp_final_call(aggsc, cntf, cw1, w1a, w1b, w2, b1, b2, gsc, beta,
                    wp, bp):
    return pl.pallas_call(
        _mlp_final_body,
        grid=(_NB,),
        in_specs=[
            pl.BlockSpec((_BN, HALF), lambda i: (i, 0)),
            pl.BlockSpec((_BN, HALF), lambda i: (_NB + i, 0)),
            pl.BlockSpec((1, _BN, 16), lambda i: (0, i, 0)),
            pl.BlockSpec((1, _BN, 16), lambda i: (1, i, 0)),
            _full((16, HID)),
            _full((HALF, HID)),
            _full((HALF, HID)),
            _full((HID, EMB)),
            _full((1, HID)),
            _full((1, EMB)),
            _full((1, EMB)),
            _full((1, EMB)),
            _full((1, EMB)),
            _full((1, 1)),
        ],
        out_specs=_full((1, 1)),
        out_shape=jax.ShapeDtypeStruct((1, 1), jnp.float32),
        scratch_shapes=[pltpu.VMEM((8, 128), jnp.float32)],
    )(aggsc, aggsc, cntf, cntf, cw1, w1a, w1b, w2, b1, b2, gsc, beta, wp, bp)


def kernel(g, x, w, node_emb0, node_emb1, edge_emb0, edge_emb1, W1, b1, W2,
           b2, gamma, beta, Wp, bp):
    f32 = jnp.float32
    src = g[0].astype(jnp.int32)
    dst = g[1].astype(jnp.int32)
    pad = E_PAD - E
    ar = jnp.arange(pad, dtype=jnp.int32)
    srcf = jnp.concatenate([src, ar % 997])
    dstf = jnp.concatenate([dst, N + (ar % TRASH)])
    idxc = jnp.concatenate([srcf.reshape(CHUNKS, 1, CH),
                            dstf.reshape(CHUNKS, 1, CH)], axis=1)
    code_e = (w[:, 0] * 3 + w[:, 1]).astype(jnp.int32)
    codep = jnp.concatenate([code_e, jnp.zeros((pad,), jnp.int32)])
    code_n = (x[:, 0] * 3 + x[:, 1]).astype(jnp.int32).reshape(_NB, 1, _BN)

    t9 = (node_emb0[:3, None, :] + node_emb1[None, :3, :]).reshape(9, EMB)
    t16 = jnp.concatenate([t9, jnp.zeros((7, EMB), f32)])
    e9 = (edge_emb0[:, :3, None, :]
          + edge_emb1[:, None, :3, :]).reshape(L, 9, EMB)
    e16 = jnp.concatenate([e9, jnp.zeros((L, 7, EMB), f32)], axis=1)

    inv_std = np.float32(1.0 / np.sqrt(1.0 + BN_EPS))

    cntf = _cnt(dstf, codep).reshape(2, NS, 16)

    h = pl.pallas_call(
        _init_body,
        grid=(_NB,),
        in_specs=[
            pl.BlockSpec((1, 1, _BN), lambda i: (i, 0, 0)),
            _full((16, EMB)),
        ],
        out_specs=pl.BlockSpec((2, _BN, HALF), lambda i: (0, i, 0)),
        out_shape=jax.ShapeDtypeStruct((2, N, HALF), f32),
    )(code_n, t16)

    out = None
    for l in range(L):
        agg = _spmv(h.reshape(2 * N, HALF), idxc)
        cw1 = jnp.dot(e16[l], W1[l])
        args = (agg, cntf, cw1,
                W1[l][:HALF], W1[l][HALF:], W2[l],
                b1[l].reshape(1, HID), b2[l].reshape(1, EMB),
                (gamma[l] * inv_std).reshape(1, EMB), beta[l].reshape(1, EMB))
        if l < L - 1:
            h = _mlp_call(*args, relu=True)
        else:
            out = _mlp_final_call(*args, Wp.reshape(1, EMB),
                                  bp.reshape(1, 1))
    return out
